# baseline (speedup 1.0000x reference)
import functools

import jax
import jax.numpy as jnp
from jax import lax
from jax.experimental import pallas as pl
from jax.experimental.pallas import tpu as pltpu
from jax.experimental.pallas import tpu_sc as plsc

F = 2048
E = 16
T = 8192
L = 16
NC = 2
NS = 16
NW = NC * NS

T_SC = 512
SPLIT = T - T_SC
TPW = T_SC // NW

CT = 512
NBUF = 4


def _tc_body(x_hbm, w_ref, b_ref, o_ref, xb, sems):
    n = SPLIT // CT

    def cp(i, s):
        return pltpu.make_async_copy(
            x_hbm.at[pl.ds(i * CT, CT)], xb.at[s], sems.at[s])

    for i in range(min(NBUF, n)):
        cp(i, i).start()
    for i in range(n):
        s = i % NBUF
        cp(i, s).wait()
        logits = lax.dot_general(
            xb[s], w_ref[...], (((1,), (1,)), ((), ())),
            preferred_element_type=jnp.float32,
        ) + b_ref[...]
        m = jnp.max(logits, axis=-1, keepdims=True)
        e = jnp.exp(logits - m)
        o_ref[pl.ds(i * CT, CT), :] = e / jnp.sum(e, axis=-1, keepdims=True)
        if i + NBUF < n:
            cp(i + NBUF, s).start()


def _tc_router(x, W, b2):
    return pl.pallas_call(
        _tc_body,
        in_specs=[
            pl.BlockSpec(memory_space=pl.ANY),
            pl.BlockSpec(memory_space=pltpu.VMEM),
            pl.BlockSpec(memory_space=pltpu.VMEM),
        ],
        out_specs=pl.BlockSpec(memory_space=pltpu.VMEM),
        out_shape=jax.ShapeDtypeStruct((SPLIT, E), jnp.float32),
        scratch_shapes=[
            pltpu.VMEM((NBUF, CT, F), jnp.float32),
            pltpu.SemaphoreType.DMA((NBUF,)),
        ],
    )(x, W, b2)


def _lane_perm(v, idx):
    return lax.gather(
        v, idx[:, None],
        dimension_numbers=lax.GatherDimensionNumbers(
            offset_dims=(), collapsed_slice_dims=(0,), start_index_map=(0,)),
        slice_sizes=(1,),
        mode=lax.GatherScatterMode.PROMISE_IN_BOUNDS,
    )


def _sc_body(x_hbm, w_hbm, b_hbm, out_hbm, wraw_v, wt_v, b_v, xb, obuf, sem_x):
    wid = lax.axis_index("s") * NC + lax.axis_index("c")
    base = wid * TPW
    xcp = pltpu.make_async_copy(
        x_hbm.at[pl.ds(SPLIT + base, TPW)], xb, sem_x)
    xcp.start()
    pltpu.sync_copy(w_hbm, wraw_v)
    pltpu.sync_copy(b_hbm, b_v)
    bvec = b_v[...]
    rowi = lax.iota(jnp.int32, L)

    def tr_body(d, idxv):
        wt_v[d, :] = plsc.load_gather(wraw_v, [idxv])
        return idxv + 1

    lax.fori_loop(0, F, tr_body, rowi * F)
    xcp.wait()

    zero = jnp.zeros((L,), jnp.float32)
    lanes = lax.iota(jnp.int32, L)
    for t in range(TPW):
        def k_body(k, accs):
            acc_a, acc_b = accs
            xv = xb[t, pl.ds(k * L, L)]
            for j in range(L):
                w = wt_v[k * L + j, :]
                if j % 2 == 0:
                    acc_a = acc_a + xv[j] * w
                else:
                    acc_b = acc_b + xv[j] * w
            return (acc_a, acc_b)

        acc_a, acc_b = lax.fori_loop(0, F // L, k_body, (bvec, zero))
        acc = acc_a + acc_b
        m = acc
        for st in (1, 2, 4, 8):
            m = jnp.maximum(m, _lane_perm(m, lanes ^ st))
        e = jnp.exp(acc - m)
        s = e
        for st in (1, 2, 4, 8):
            s = s + _lane_perm(s, lanes ^ st)
        obuf[t, :] = e / s

    pltpu.sync_copy(obuf, out_hbm.at[pl.ds(base, TPW)])


def _sc_router(x, W, b):
    mesh = plsc.VectorSubcoreMesh(core_axis_name="c", subcore_axis_name="s")
    return functools.partial(
        pl.kernel,
        out_type=jax.ShapeDtypeStruct((T_SC, E), jnp.float32),
        mesh=mesh,
        scratch_types=[
            pltpu.VMEM((E * F,), jnp.float32),
            pltpu.VMEM((F, E), jnp.float32),
            pltpu.VMEM((L,), jnp.float32),
            pltpu.VMEM((TPW, F), jnp.float32),
            pltpu.VMEM((TPW, E), jnp.float32),
            pltpu.SemaphoreType.DMA,
        ],
        compiler_params=pltpu.CompilerParams(use_tc_tiling_on_sc=False, needs_layout_passes=False),
    )(_sc_body)(x, W.reshape(-1), b)


def kernel(x, W, b):
    sc_out = _sc_router(x, W, b)
    tc_out = _tc_router(x, W, b.reshape(1, E))
    return jnp.concatenate([tc_out, sc_out], axis=0)


# speedup vs baseline: 4.3071x; 4.3071x over previous
import jax
import jax.numpy as jnp
from jax import lax
from jax.experimental import pallas as pl
from jax.experimental.pallas import tpu as pltpu

F = 2048
E = 16
T = 8192
CT = 512
NBUF = 4


def _tc_body(x_hbm, w_ref, b_ref, o_ref, xb, sems):
    n = T // CT

    def cp(i, s):
        return pltpu.make_async_copy(
            x_hbm.at[pl.ds(i * CT, CT)], xb.at[s], sems.at[s])

    for i in range(min(NBUF, n)):
        cp(i, i).start()
    for i in range(n):
        s = i % NBUF
        cp(i, s).wait()
        logits = lax.dot_general(
            xb[s], w_ref[...], (((1,), (1,)), ((), ())),
            preferred_element_type=jnp.float32,
        ) + b_ref[...]
        m = jnp.max(logits, axis=-1, keepdims=True)
        e = jnp.exp(logits - m)
        o_ref[pl.ds(i * CT, CT), :] = e / jnp.sum(e, axis=-1, keepdims=True)
        if i + NBUF < n:
            cp(i + NBUF, s).start()


def kernel(x, W, b):
    return pl.pallas_call(
        _tc_body,
        in_specs=[
            pl.BlockSpec(memory_space=pl.ANY),
            pl.BlockSpec(memory_space=pltpu.VMEM),
            pl.BlockSpec(memory_space=pltpu.VMEM),
        ],
        out_specs=pl.BlockSpec(memory_space=pltpu.VMEM),
        out_shape=jax.ShapeDtypeStruct((T, E), jnp.float32),
        scratch_shapes=[
            pltpu.VMEM((NBUF, CT, F), jnp.float32),
            pltpu.SemaphoreType.DMA((NBUF,)),
        ],
    )(x, W, b.reshape(1, E))
